# baseline (speedup 1.0000x reference)
import functools

import numpy as np
import jax
import jax.numpy as jnp
from jax.experimental import pallas as pl
from jax.experimental.pallas import tpu as pltpu
from jax.experimental.pallas import tpu_sc as plsc

_B, _T, _L, _D, _H, _E = 2, 4, 2048, 768, 12, 64
_U = 40
_IDX_SAMPLE = np.random.default_rng(0).choice(_L, _U, replace=False)
_CH = 512
_NC = _L // _CH
_NEG = float("-inf")
_HIGH = jax.lax.Precision.HIGHEST


def _proj_kernel(q_ref, v_ref, vs_ref, wq_ref, wv_ref, wrow_ref,
                 vout_ref, qout_ref, idx_ref, fi_ref, ob_ref, m_scr,
                 carry_scr):
    c = pl.program_id(1)
    prec_m = None

    Qc = jnp.dot(q_ref[0, 0], wq_ref[...], preferred_element_type=jnp.float32,
                 precision=prec_m)
    Vc = jnp.dot(v_ref[0, 0], wv_ref[...], preferred_element_type=jnp.float32,
                 precision=prec_m)
    vout_ref[0, 0] = Vc
    qout_ref[0, 0] = Qc

    Ks = jnp.dot(vs_ref[0, 0], wv_ref[...], preferred_element_type=jnp.float32,
                 precision=prec_m)

    @pl.when(c == 0)
    def _():
        carry_scr[...] = jnp.zeros_like(carry_scr)

    for h in range(_H):
        Ksh = Ks[:, h * _E:(h + 1) * _E]
        Qh = Qc[:, h * _E:(h + 1) * _E]
        St = jax.lax.dot_general(Ksh, Qh, (((1,), (1,)), ((), ())),
                                 preferred_element_type=jnp.float32,
                                 precision=prec_m)
        Mrow = jnp.max(St, axis=0) - jnp.sum(St, axis=0) * (1.0 / _L)
        m_scr[h, pl.ds(c * _CH, _CH)] = Mrow

    Vw = Vc * wrow_ref[0:1, :]
    R = (Vw[:, 0:128] + Vw[:, 128:256] + Vw[:, 256:384] + Vw[:, 384:512]
         + Vw[:, 512:640] + Vw[:, 640:768])
    Z = R[:, 0:_E] + R[:, _E:128]
    tri = (jax.lax.broadcasted_iota(jnp.int32, (_CH, _CH), 0)
           >= jax.lax.broadcasted_iota(jnp.int32, (_CH, _CH), 1)
           ).astype(jnp.float32)
    cumz = jnp.dot(tri, Z, preferred_element_type=jnp.float32,
                   precision=_HIGH) + carry_scr[0:1, :]
    carry_scr[0:1, :] = carry_scr[0:1, :] + jnp.sum(Z, axis=0, keepdims=True)
    ob_ref[0, 0] = cumz

    @pl.when(c == _NC - 1)
    def _():
        rows = jax.lax.broadcasted_iota(jnp.int32, (16, _L), 0)
        lanes = jax.lax.broadcasted_iota(jnp.int32, (16, _L), 1)
        cols = jax.lax.broadcasted_iota(jnp.int32, (16, 128), 1)
        M = jnp.where(rows < _H, m_scr[...], _NEG)

        def body(j, state):
            M, acc = state
            mval = jnp.max(M, axis=1, keepdims=True)
            cand = jnp.where(M == mval, lanes, _L)
            idx = jnp.min(cand, axis=1, keepdims=True)
            acc = jnp.where(cols == j, idx, acc)
            M = jnp.where(lanes == idx, _NEG, M)
            return M, acc

        _, acc = jax.lax.fori_loop(0, _U, body,
                                   (M, jnp.zeros((16, 128), jnp.int32)))
        idx_ref[0, 0] = acc
        t = pl.program_id(0) % _T
        rows128 = jax.lax.broadcasted_iota(jnp.int32, (16, 128), 0)
        flat = (t * _L + acc) * _H + rows128
        fi_ref[0, 0] = flat[:_H, :_U]


_NW = 32
_NG = _B * _T * _H * _U
_PER_W = _NG // _NW


def _sc_gather(qv, idxflat):
    mesh = plsc.VectorSubcoreMesh(core_axis_name="c", subcore_axis_name="s")

    @functools.partial(
        pl.kernel, mesh=mesh,
        compiler_params=pltpu.CompilerParams(use_tc_tiling_on_sc=False),
        out_type=jax.ShapeDtypeStruct((_NG, _E), jnp.float32),
        scratch_types=[
            pltpu.VMEM((_PER_W,), jnp.int32),
            pltpu.VMEM((_PER_W, _E), jnp.float32),
            pltpu.SemaphoreType.DMA,
        ],
    )
    def k(q_hbm, idx_hbm, out_hbm, idx_v, rows_v, sem):
        wid = jax.lax.axis_index("s") * 2 + jax.lax.axis_index("c")
        base = wid * _PER_W
        pltpu.sync_copy(idx_hbm.at[pl.ds(base, _PER_W)], idx_v)
        pltpu.async_copy(q_hbm.at[idx_v], rows_v, sem).wait()
        pltpu.sync_copy(rows_v, out_hbm.at[pl.ds(base, _PER_W)])

    return k(qv, idxflat)


def _fix_kernel(v0_ref, v1_ref, qr0_ref, qr1_ref, i0_ref, i1_ref, ob_ref,
                w_ref, s_ref, out_ref):
    g = pl.program_id(1)
    V0 = v0_ref[0, 0]
    V1 = v1_ref[0, 0]
    tau = s_ref[0]
    delta = s_ref[1]
    scale = jnp.float32(1.0 / np.sqrt(_E))
    lanesL = jax.lax.broadcasted_iota(jnp.int32, (_U, _L), 1)

    acc = jnp.zeros((_L, _E), jnp.float32)
    for hh in range(2):
        h = 2 * g + hh
        wh = w_ref[h]
        sl = slice(hh * _E, (hh + 1) * _E)
        V0h = V0[:, sl]
        Ps, deltas = [], []
        for Vh, iref, qref in ((V0h, i0_ref, qr0_ref), (V1, i1_ref, qr1_ref)):
            if Vh is V1:
                Vh = V1[:, sl]
            idx_col = iref[0, 0, pl.ds(h, 1), :][0, :_U].reshape(_U, 1)
            P = (lanesL == idx_col).astype(jnp.float32)
            SM = (lanesL <= idx_col).astype(jnp.float32)
            Qr = qref[0, 0, pl.ds(h, 1)][0]
            S = jax.lax.dot_general(Qr, Vh, (((1,), (1,)), ((), ())),
                                    preferred_element_type=jnp.float32)
            S = (S * tau + delta) * scale
            S = S - jnp.max(S, axis=1, keepdims=True)
            Sexp = jnp.exp(S)
            A = Sexp / jnp.sum(Sexp, axis=1, keepdims=True)
            attn = jnp.dot(A, Vh, preferred_element_type=jnp.float32)
            cumsel = jnp.dot(SM, V0h, preferred_element_type=jnp.float32,
                             precision=_HIGH)
            Ps.append(P)
            deltas.append(attn - cumsel)
        m1 = jnp.sum(Ps[1], axis=0)
        t0 = jax.lax.dot_general(Ps[0], deltas[0], (((0,), (0,)), ((), ())),
                                 preferred_element_type=jnp.float32)
        t1 = jax.lax.dot_general(Ps[1], deltas[1], (((0,), (0,)), ((), ())),
                                 preferred_element_type=jnp.float32)
        acc = acc + wh * (t0 * (1.0 - m1)[:, None] + t1)

    @pl.when(g == 0)
    def _():
        out_ref[0] = ob_ref[0, 0]
    out_ref[0] += acc


def kernel(queries, keys, values, Wq, Wk, Wv, w_out, tau, delta):
    del keys, Wk
    wq2 = Wq.reshape(_D, _H * _E)
    wv2 = Wv.reshape(_D, _H * _E)
    vs = values[:, :, _IDX_SAMPLE, :]
    wrow = jnp.broadcast_to(jnp.repeat(w_out, _E)[None, :], (8, _H * _E))
    w_pad = jnp.concatenate([w_out, jnp.zeros((4,), jnp.float32)])
    scl = jnp.concatenate([tau, delta]).astype(jnp.float32)

    def bmap(p):
        return 1 - p // _T

    def tmap(p):
        return p % _T

    V, Q, topidx, flatidx, out_base = pl.pallas_call(
        _proj_kernel,
        grid=(_B * _T, _NC),
        in_specs=[
            pl.BlockSpec((1, 1, _CH, _D), lambda p, c: (bmap(p), tmap(p), c, 0)),
            pl.BlockSpec((1, 1, _CH, _D), lambda p, c: (bmap(p), tmap(p), c, 0)),
            pl.BlockSpec((1, 1, _U, _D), lambda p, c: (bmap(p), tmap(p), 0, 0)),
            pl.BlockSpec((_D, _H * _E), lambda p, c: (0, 0)),
            pl.BlockSpec((_D, _H * _E), lambda p, c: (0, 0)),
            pl.BlockSpec((8, _H * _E), lambda p, c: (0, 0)),
        ],
        out_specs=[
            pl.BlockSpec((1, 1, _CH, _H * _E),
                         lambda p, c: (bmap(p), tmap(p), c, 0)),
            pl.BlockSpec((1, 1, _CH, _H * _E),
                         lambda p, c: (bmap(p), tmap(p), c, 0)),
            pl.BlockSpec((1, 1, 16, 128), lambda p, c: (bmap(p), tmap(p), 0, 0)),
            pl.BlockSpec((1, 1, _H, _U), lambda p, c: (bmap(p), tmap(p), 0, 0)),
            pl.BlockSpec((1, 1, _CH, _E), lambda p, c: (bmap(p), tmap(p), c, 0)),
        ],
        out_shape=[
            jax.ShapeDtypeStruct((_B, _T, _L, _H * _E), jnp.float32),
            jax.ShapeDtypeStruct((_B, _T, _L, _H * _E), jnp.float32),
            jax.ShapeDtypeStruct((_B, _T, 16, 128), jnp.int32),
            jax.ShapeDtypeStruct((_B, _T, _H, _U), jnp.int32),
            jax.ShapeDtypeStruct((_B, _T, _L, _E), jnp.float32),
        ],
        scratch_shapes=[
            pltpu.VMEM((16, _L), jnp.float32),
            pltpu.VMEM((8, _E), jnp.float32),
        ],
    )(queries, values, vs, wq2, wv2, wrow)

    qv = Q[0].reshape(_T * _L * _H, _E)
    qr = _sc_gather(qv, flatidx.reshape(_NG)).reshape(_B, _T, _H, _U, _E)

    out0 = pl.pallas_call(
        _fix_kernel,
        grid=(_T, _H // 2),
        in_specs=[
            pl.BlockSpec((1, 1, _L, 128), lambda t, g: (0, t, 0, g)),
            pl.BlockSpec((1, 1, _L, 128), lambda t, g: (1, t, 0, g)),
            pl.BlockSpec((1, 1, _H, _U, _E), lambda t, g: (0, t, 0, 0, 0)),
            pl.BlockSpec((1, 1, _H, _U, _E), lambda t, g: (1, t, 0, 0, 0)),
            pl.BlockSpec((1, 1, 16, 128), lambda t, g: (0, t, 0, 0)),
            pl.BlockSpec((1, 1, 16, 128), lambda t, g: (1, t, 0, 0)),
            pl.BlockSpec((1, 1, _L, _E), lambda t, g: (0, t, 0, 0)),
            pl.BlockSpec(memory_space=pltpu.SMEM),
            pl.BlockSpec(memory_space=pltpu.SMEM),
        ],
        out_specs=pl.BlockSpec((1, _L, _E), lambda t, g: (t, 0, 0)),
        out_shape=jax.ShapeDtypeStruct((_T, _L, _E), jnp.float32),
    )(V, V, qr, qr, topidx, topidx, out_base, w_pad, scl)

    return jnp.concatenate([out0[None], out_base[1:]], axis=0)


# speedup vs baseline: 1.2700x; 1.2700x over previous
import functools

import numpy as np
import jax
import jax.numpy as jnp
from jax.experimental import pallas as pl
from jax.experimental.pallas import tpu as pltpu
from jax.experimental.pallas import tpu_sc as plsc

_B, _T, _L, _D, _H, _E = 2, 4, 2048, 768, 12, 64
_U = 40
_IDX_SAMPLE = np.random.default_rng(0).choice(_L, _U, replace=False)
_CH = 512
_NC = _L // _CH
_NEG = float("-inf")
_HIGH = jax.lax.Precision.HIGHEST


def _proj_kernel(q_ref, v_ref, vs_ref, wq_ref, wv_ref, wrow_ref,
                 vout_ref, qout_ref, idx_ref, fi_ref, ob_ref, m_scr,
                 carry_scr):
    c = pl.program_id(1)
    prec_m = None

    Qc = jnp.dot(q_ref[0, 0], wq_ref[...], preferred_element_type=jnp.float32,
                 precision=prec_m)
    Vc = jnp.dot(v_ref[0, 0], wv_ref[...], preferred_element_type=jnp.float32,
                 precision=prec_m)
    vout_ref[0, 0] = Vc
    qout_ref[0, 0] = Qc

    Ks = jnp.dot(vs_ref[0, 0], wv_ref[...], preferred_element_type=jnp.float32,
                 precision=prec_m)

    @pl.when(c == 0)
    def _():
        carry_scr[...] = jnp.zeros_like(carry_scr)

    for h in range(_H):
        Ksh = Ks[:, h * _E:(h + 1) * _E]
        Qh = Qc[:, h * _E:(h + 1) * _E]
        St = jax.lax.dot_general(Ksh, Qh, (((1,), (1,)), ((), ())),
                                 preferred_element_type=jnp.float32,
                                 precision=prec_m)
        Mrow = jnp.max(St, axis=0) - jnp.sum(St, axis=0) * (1.0 / _L)
        m_scr[h, pl.ds(c * _CH, _CH)] = Mrow

    Vw = Vc * wrow_ref[0:1, :]
    R = (Vw[:, 0:128] + Vw[:, 128:256] + Vw[:, 256:384] + Vw[:, 384:512]
         + Vw[:, 512:640] + Vw[:, 640:768])
    Z = R[:, 0:_E] + R[:, _E:128]
    tri = (jax.lax.broadcasted_iota(jnp.int32, (_CH, _CH), 0)
           >= jax.lax.broadcasted_iota(jnp.int32, (_CH, _CH), 1)
           ).astype(jnp.float32)
    cumz = jnp.dot(tri, Z, preferred_element_type=jnp.float32,
                   precision=None) + carry_scr[0:1, :]
    carry_scr[0:1, :] = carry_scr[0:1, :] + jnp.sum(Z, axis=0, keepdims=True)
    ob_ref[0, 0] = cumz

    @pl.when(c == _NC - 1)
    def _():
        rows = jax.lax.broadcasted_iota(jnp.int32, (16, _L), 0)
        lanes = jax.lax.broadcasted_iota(jnp.int32, (16, _L), 1)
        cols = jax.lax.broadcasted_iota(jnp.int32, (16, 128), 1)
        M = jnp.where(rows < _H, m_scr[...], _NEG)

        def body(j, state):
            M, acc = state
            mval = jnp.max(M, axis=1, keepdims=True)
            cand = jnp.where(M == mval, lanes, _L)
            idx = jnp.min(cand, axis=1, keepdims=True)
            acc = jnp.where(cols == j, idx, acc)
            M = jnp.where(lanes == idx, _NEG, M)
            return M, acc

        _, acc = jax.lax.fori_loop(0, _U, body,
                                   (M, jnp.zeros((16, 128), jnp.int32)))
        idx_ref[0, 0] = acc
        t = pl.program_id(0) % _T
        rows128 = jax.lax.broadcasted_iota(jnp.int32, (16, 128), 0)
        flat = (t * _L + acc) * _H + rows128
        fi_ref[0, 0] = flat[:_H, :_U]


_NW = 32
_NG = _B * _T * _H * _U
_PER_W = _NG // _NW


def _sc_gather(qv, idxflat):
    mesh = plsc.VectorSubcoreMesh(core_axis_name="c", subcore_axis_name="s")

    @functools.partial(
        pl.kernel, mesh=mesh,
        compiler_params=pltpu.CompilerParams(use_tc_tiling_on_sc=False),
        out_type=jax.ShapeDtypeStruct((_NG, _E), jnp.float32),
        scratch_types=[
            pltpu.VMEM((_PER_W,), jnp.int32),
            pltpu.VMEM((_PER_W, _E), jnp.float32),
            pltpu.SemaphoreType.DMA,
        ],
    )
    def k(q_hbm, idx_hbm, out_hbm, idx_v, rows_v, sem):
        wid = jax.lax.axis_index("s") * 2 + jax.lax.axis_index("c")
        base = wid * _PER_W
        pltpu.sync_copy(idx_hbm.at[pl.ds(base, _PER_W)], idx_v)
        pltpu.async_copy(q_hbm.at[idx_v], rows_v, sem).wait()
        pltpu.sync_copy(rows_v, out_hbm.at[pl.ds(base, _PER_W)])

    return k(qv, idxflat)


def _fix_kernel(v0_ref, v1_ref, qr0_ref, qr1_ref, i0_ref, i1_ref, ob_ref,
                w_ref, s_ref, out_ref):
    g = pl.program_id(1)
    V0 = v0_ref[0, 0]
    V1 = v1_ref[0, 0]
    tau = s_ref[0]
    delta = s_ref[1]
    scale = jnp.float32(1.0 / np.sqrt(_E))
    lanesL = jax.lax.broadcasted_iota(jnp.int32, (_U, _L), 1)

    acc = jnp.zeros((_L, _E), jnp.float32)
    for hh in range(2):
        h = 2 * g + hh
        wh = w_ref[h]
        sl = slice(hh * _E, (hh + 1) * _E)
        V0h = V0[:, sl]
        Ps, deltas = [], []
        for Vh, iref, qref in ((V0h, i0_ref, qr0_ref), (V1, i1_ref, qr1_ref)):
            if Vh is V1:
                Vh = V1[:, sl]
            idx_col = iref[0, 0, pl.ds(h, 1), :][0, :_U].reshape(_U, 1)
            P = (lanesL == idx_col).astype(jnp.float32)
            SM = (lanesL <= idx_col).astype(jnp.float32)
            Qr = qref[0, 0, pl.ds(h, 1)][0]
            S = jax.lax.dot_general(Qr, Vh, (((1,), (1,)), ((), ())),
                                    preferred_element_type=jnp.float32)
            S = (S * tau + delta) * scale
            S = S - jnp.max(S, axis=1, keepdims=True)
            Sexp = jnp.exp(S)
            A = Sexp / jnp.sum(Sexp, axis=1, keepdims=True)
            attn = jnp.dot(A, Vh, preferred_element_type=jnp.float32)
            cumsel = jnp.dot(SM, V0h, preferred_element_type=jnp.float32,
                             precision=None)
            Ps.append(P)
            deltas.append(attn - cumsel)
        m1 = jnp.sum(Ps[1], axis=0)
        t0 = jax.lax.dot_general(Ps[0], deltas[0], (((0,), (0,)), ((), ())),
                                 preferred_element_type=jnp.float32)
        t1 = jax.lax.dot_general(Ps[1], deltas[1], (((0,), (0,)), ((), ())),
                                 preferred_element_type=jnp.float32)
        acc = acc + wh * (t0 * (1.0 - m1)[:, None] + t1)

    @pl.when(g == 0)
    def _():
        out_ref[0] = ob_ref[0, 0]
    out_ref[0] += acc


def kernel(queries, keys, values, Wq, Wk, Wv, w_out, tau, delta):
    del keys, Wk
    wq2 = Wq.reshape(_D, _H * _E)
    wv2 = Wv.reshape(_D, _H * _E)
    vs = values[:, :, _IDX_SAMPLE, :]
    wrow = jnp.broadcast_to(jnp.repeat(w_out, _E)[None, :], (8, _H * _E))
    w_pad = jnp.concatenate([w_out, jnp.zeros((4,), jnp.float32)])
    scl = jnp.concatenate([tau, delta]).astype(jnp.float32)

    def bmap(p):
        return 1 - p // _T

    def tmap(p):
        return p % _T

    V, Q, topidx, flatidx, out_base = pl.pallas_call(
        _proj_kernel,
        grid=(_B * _T, _NC),
        in_specs=[
            pl.BlockSpec((1, 1, _CH, _D), lambda p, c: (bmap(p), tmap(p), c, 0)),
            pl.BlockSpec((1, 1, _CH, _D), lambda p, c: (bmap(p), tmap(p), c, 0)),
            pl.BlockSpec((1, 1, _U, _D), lambda p, c: (bmap(p), tmap(p), 0, 0)),
            pl.BlockSpec((_D, _H * _E), lambda p, c: (0, 0)),
            pl.BlockSpec((_D, _H * _E), lambda p, c: (0, 0)),
            pl.BlockSpec((8, _H * _E), lambda p, c: (0, 0)),
        ],
        out_specs=[
            pl.BlockSpec((1, 1, _CH, _H * _E),
                         lambda p, c: (bmap(p), tmap(p), c, 0)),
            pl.BlockSpec((1, 1, _CH, _H * _E),
                         lambda p, c: (bmap(p), tmap(p), c, 0)),
            pl.BlockSpec((1, 1, 16, 128), lambda p, c: (bmap(p), tmap(p), 0, 0)),
            pl.BlockSpec((1, 1, _H, _U), lambda p, c: (bmap(p), tmap(p), 0, 0)),
            pl.BlockSpec((1, 1, _CH, _E), lambda p, c: (bmap(p), tmap(p), c, 0)),
        ],
        out_shape=[
            jax.ShapeDtypeStruct((_B, _T, _L, _H * _E), jnp.float32),
            jax.ShapeDtypeStruct((_B, _T, _L, _H * _E), jnp.float32),
            jax.ShapeDtypeStruct((_B, _T, 16, 128), jnp.int32),
            jax.ShapeDtypeStruct((_B, _T, _H, _U), jnp.int32),
            jax.ShapeDtypeStruct((_B, _T, _L, _E), jnp.float32),
        ],
        scratch_shapes=[
            pltpu.VMEM((16, _L), jnp.float32),
            pltpu.VMEM((8, _E), jnp.float32),
        ],
    )(queries, values, vs, wq2, wv2, wrow)

    qv = Q[0].reshape(_T * _L * _H, _E)
    qr = _sc_gather(qv, flatidx.reshape(_NG)).reshape(_B, _T, _H, _U, _E)

    out0 = pl.pallas_call(
        _fix_kernel,
        grid=(_T, _H // 2),
        in_specs=[
            pl.BlockSpec((1, 1, _L, 128), lambda t, g: (0, t, 0, g)),
            pl.BlockSpec((1, 1, _L, 128), lambda t, g: (1, t, 0, g)),
            pl.BlockSpec((1, 1, _H, _U, _E), lambda t, g: (0, t, 0, 0, 0)),
            pl.BlockSpec((1, 1, _H, _U, _E), lambda t, g: (1, t, 0, 0, 0)),
            pl.BlockSpec((1, 1, 16, 128), lambda t, g: (0, t, 0, 0)),
            pl.BlockSpec((1, 1, 16, 128), lambda t, g: (1, t, 0, 0)),
            pl.BlockSpec((1, 1, _L, _E), lambda t, g: (0, t, 0, 0)),
            pl.BlockSpec(memory_space=pltpu.SMEM),
            pl.BlockSpec(memory_space=pltpu.SMEM),
        ],
        out_specs=pl.BlockSpec((1, _L, _E), lambda t, g: (t, 0, 0)),
        out_shape=jax.ShapeDtypeStruct((_T, _L, _E), jnp.float32),
    )(V, V, qr, qr, topidx, topidx, out_base, w_pad, scl)

    return jnp.concatenate([out0[None], out_base[1:]], axis=0)


# speedup vs baseline: 1.4395x; 1.1334x over previous
import functools

import numpy as np
import jax
import jax.numpy as jnp
from jax.experimental import pallas as pl
from jax.experimental.pallas import tpu as pltpu
from jax.experimental.pallas import tpu_sc as plsc

_B, _T, _L, _D, _H, _E = 2, 4, 2048, 768, 12, 64
_U = 40
_IDX_SAMPLE = np.random.default_rng(0).choice(_L, _U, replace=False)
_CH = 512
_NC = _L // _CH
_NEG = float("-inf")
_HIGH = jax.lax.Precision.HIGHEST


def _proj_kernel(q_ref, v_ref, vs_ref, wq_ref, wv_ref, wrow_ref,
                 vout_ref, qout_ref, idx_ref, fi_ref, ob_ref, m_scr,
                 carry_scr, ksbd_scr):
    c = pl.program_id(1)
    prec_m = None

    Qc = jnp.dot(q_ref[0, 0], wq_ref[...], preferred_element_type=jnp.float32,
                 precision=prec_m)
    Vc = jnp.dot(v_ref[0, 0], wv_ref[...], preferred_element_type=jnp.float32,
                 precision=prec_m)
    vout_ref[0, 0] = Vc
    qout_ref[0, 0] = Qc

    @pl.when(c == 0)
    def _():
        carry_scr[...] = jnp.zeros_like(carry_scr)
        Ks = jnp.dot(vs_ref[0, 0], wv_ref[...],
                     preferred_element_type=jnp.float32,
                     precision=prec_m)
        rowsb = jax.lax.broadcasted_iota(jnp.int32, (_H * _U, _H * _E), 0)
        colsb = jax.lax.broadcasted_iota(jnp.int32, (_H * _U, _H * _E), 1)
        Ktile = jnp.concatenate([Ks] * _H, axis=0)
        ksbd_scr[...] = jnp.where(rowsb // _U == colsb // _E, Ktile, 0.0)

    St = jax.lax.dot_general(ksbd_scr[...], Qc, (((1,), (1,)), ((), ())),
                             preferred_element_type=jnp.float32,
                             precision=prec_m)
    for h in range(_H):
        Sh = St[h * _U:(h + 1) * _U, :]
        Mrow = jnp.max(Sh, axis=0) - jnp.sum(Sh, axis=0) * (1.0 / _L)
        m_scr[h, pl.ds(c * _CH, _CH)] = Mrow

    Vw = Vc * wrow_ref[0:1, :]
    R = (Vw[:, 0:128] + Vw[:, 128:256] + Vw[:, 256:384] + Vw[:, 384:512]
         + Vw[:, 512:640] + Vw[:, 640:768])
    Z = R[:, 0:_E] + R[:, _E:128]
    tri = (jax.lax.broadcasted_iota(jnp.int32, (_CH, _CH), 0)
           >= jax.lax.broadcasted_iota(jnp.int32, (_CH, _CH), 1)
           ).astype(jnp.float32)
    cumz = jnp.dot(tri, Z, preferred_element_type=jnp.float32,
                   precision=None) + carry_scr[0:1, :]
    carry_scr[0:1, :] = carry_scr[0:1, :] + jnp.sum(Z, axis=0, keepdims=True)
    ob_ref[0, 0] = cumz

    @pl.when(c == _NC - 1)
    def _():
        rows = jax.lax.broadcasted_iota(jnp.int32, (16, _L), 0)
        lanes = jax.lax.broadcasted_iota(jnp.int32, (16, _L), 1)
        cols = jax.lax.broadcasted_iota(jnp.int32, (16, 128), 1)
        M = jnp.where(rows < _H, m_scr[...], _NEG)

        def body(j, state):
            M, acc = state
            mval = jnp.max(M, axis=1, keepdims=True)
            cand = jnp.where(M == mval, lanes, _L)
            idx = jnp.min(cand, axis=1, keepdims=True)
            acc = jnp.where(cols == j, idx, acc)
            M = jnp.where(lanes == idx, _NEG, M)
            return M, acc

        _, acc = jax.lax.fori_loop(0, _U, body,
                                   (M, jnp.zeros((16, 128), jnp.int32)))
        idx_ref[0, 0] = acc
        t = pl.program_id(0) % _T
        rows128 = jax.lax.broadcasted_iota(jnp.int32, (16, 128), 0)
        flat = (t * _L + acc) * _H + rows128
        fi_ref[0, 0] = flat[:_H, :_U]


_NW = 32
_NG = _B * _T * _H * _U
_PER_W = _NG // _NW


def _sc_gather(qv, idxflat):
    mesh = plsc.VectorSubcoreMesh(core_axis_name="c", subcore_axis_name="s")

    @functools.partial(
        pl.kernel, mesh=mesh,
        compiler_params=pltpu.CompilerParams(use_tc_tiling_on_sc=False),
        out_type=jax.ShapeDtypeStruct((_NG, _E), jnp.float32),
        scratch_types=[
            pltpu.VMEM((_PER_W,), jnp.int32),
            pltpu.VMEM((_PER_W, _E), jnp.float32),
            pltpu.SemaphoreType.DMA,
        ],
    )
    def k(q_hbm, idx_hbm, out_hbm, idx_v, rows_v, sem):
        wid = jax.lax.axis_index("s") * 2 + jax.lax.axis_index("c")
        base = wid * _PER_W
        pltpu.sync_copy(idx_hbm.at[pl.ds(base, _PER_W)], idx_v)
        pltpu.async_copy(q_hbm.at[idx_v], rows_v, sem).wait()
        pltpu.sync_copy(rows_v, out_hbm.at[pl.ds(base, _PER_W)])

    return k(qv, idxflat)


def _fix_kernel(v0_ref, v1_ref, qr0_ref, qr1_ref, i0_ref, i1_ref, ob_ref,
                w_ref, s_ref, out_ref):
    g = pl.program_id(1)
    V0 = v0_ref[0, 0]
    V1 = v1_ref[0, 0]
    tau = s_ref[0]
    delta = s_ref[1]
    scale = jnp.float32(1.0 / np.sqrt(_E))
    U2 = 2 * _U
    lanesL2 = jax.lax.broadcasted_iota(jnp.int32, (U2, _L), 1)
    rows2 = jax.lax.broadcasted_iota(jnp.int32, (U2, 128), 0)
    cols2 = jax.lax.broadcasted_iota(jnp.int32, (U2, 128), 1)
    bdmask = ((rows2 < _U) == (cols2 < _E)).astype(jnp.float32)
    w2 = jnp.where(rows2[:, 0:1] < _U, w_ref[2 * g], w_ref[2 * g + 1])

    idx2, P2, SM2, Qr2 = [], [], [], []
    for iref, qref in ((i0_ref, qr0_ref), (i1_ref, qr1_ref)):
        ia = iref[0, 0, pl.ds(2 * g, 1), :][0, :_U].reshape(_U, 1)
        ib = iref[0, 0, pl.ds(2 * g + 1, 1), :][0, :_U].reshape(_U, 1)
        ix = jnp.concatenate([ia, ib], axis=0)
        idx2.append(ix)
        P2.append((lanesL2 == ix).astype(jnp.float32))
        SM2.append((lanesL2 <= ix).astype(jnp.float32))
        qp = qref[0, 0, pl.ds(2 * g, 2)]
        qa, qb = qp[0], qp[1]
        z = jnp.zeros((_U, _E), jnp.float32)
        Qr2.append(jnp.concatenate([
            jnp.concatenate([qa, z], axis=1),
            jnp.concatenate([z, qb], axis=1)], axis=0))

    deltas = []
    for b, Vp in enumerate((V0, V1)):
        S = jax.lax.dot_general(Qr2[b], Vp, (((1,), (1,)), ((), ())),
                                preferred_element_type=jnp.float32)
        S = (S * tau + delta) * scale
        S = S - jnp.max(S, axis=1, keepdims=True)
        Sexp = jnp.exp(S)
        A = Sexp / jnp.sum(Sexp, axis=1, keepdims=True)
        attn = jnp.dot(A, Vp, preferred_element_type=jnp.float32)
        cumsel = jnp.dot(SM2[b], V0, preferred_element_type=jnp.float32)
        deltas.append((attn - cumsel) * bdmask * w2)

    eq = (idx2[0] == idx2[1].reshape(1, U2)).astype(jnp.float32)
    same_head = ((jax.lax.broadcasted_iota(jnp.int32, (U2, U2), 0) < _U)
                 == (jax.lax.broadcasted_iota(jnp.int32, (U2, U2), 1) < _U)
                 ).astype(jnp.float32)
    hit = jnp.max(eq * same_head, axis=1, keepdims=True)
    d0 = deltas[0] * (1.0 - hit)

    Pall = jnp.concatenate([P2[0], P2[1]], axis=0)
    dall = jnp.concatenate([d0, deltas[1]], axis=0)
    tpair = jax.lax.dot_general(Pall, dall, (((0,), (0,)), ((), ())),
                                preferred_element_type=jnp.float32)
    acc = tpair[:, 0:_E] + tpair[:, _E:128]

    @pl.when(g == 0)
    def _():
        out_ref[0] = ob_ref[0, 0]
    out_ref[0] += acc


def kernel(queries, keys, values, Wq, Wk, Wv, w_out, tau, delta):
    del keys, Wk
    wq2 = Wq.reshape(_D, _H * _E)
    wv2 = Wv.reshape(_D, _H * _E)
    vs = values[:, :, _IDX_SAMPLE, :]
    wrow = jnp.broadcast_to(jnp.repeat(w_out, _E)[None, :], (8, _H * _E))
    w_pad = jnp.concatenate([w_out, jnp.zeros((4,), jnp.float32)])
    scl = jnp.concatenate([tau, delta]).astype(jnp.float32)

    def bmap(p):
        return 1 - p // _T

    def tmap(p):
        return p % _T

    V, Q, topidx, flatidx, out_base = pl.pallas_call(
        _proj_kernel,
        grid=(_B * _T, _NC),
        in_specs=[
            pl.BlockSpec((1, 1, _CH, _D), lambda p, c: (bmap(p), tmap(p), c, 0)),
            pl.BlockSpec((1, 1, _CH, _D), lambda p, c: (bmap(p), tmap(p), c, 0)),
            pl.BlockSpec((1, 1, _U, _D), lambda p, c: (bmap(p), tmap(p), 0, 0)),
            pl.BlockSpec((_D, _H * _E), lambda p, c: (0, 0)),
            pl.BlockSpec((_D, _H * _E), lambda p, c: (0, 0)),
            pl.BlockSpec((8, _H * _E), lambda p, c: (0, 0)),
        ],
        out_specs=[
            pl.BlockSpec((1, 1, _CH, _H * _E),
                         lambda p, c: (bmap(p), tmap(p), c, 0)),
            pl.BlockSpec((1, 1, _CH, _H * _E),
                         lambda p, c: (bmap(p), tmap(p), c, 0)),
            pl.BlockSpec((1, 1, 16, 128), lambda p, c: (bmap(p), tmap(p), 0, 0)),
            pl.BlockSpec((1, 1, _H, _U), lambda p, c: (bmap(p), tmap(p), 0, 0)),
            pl.BlockSpec((1, 1, _CH, _E), lambda p, c: (bmap(p), tmap(p), c, 0)),
        ],
        out_shape=[
            jax.ShapeDtypeStruct((_B, _T, _L, _H * _E), jnp.float32),
            jax.ShapeDtypeStruct((_B, _T, _L, _H * _E), jnp.float32),
            jax.ShapeDtypeStruct((_B, _T, 16, 128), jnp.int32),
            jax.ShapeDtypeStruct((_B, _T, _H, _U), jnp.int32),
            jax.ShapeDtypeStruct((_B, _T, _L, _E), jnp.float32),
        ],
        scratch_shapes=[
            pltpu.VMEM((16, _L), jnp.float32),
            pltpu.VMEM((8, _E), jnp.float32),
            pltpu.VMEM((_H * _U, _H * _E), jnp.float32),
        ],
    )(queries, values, vs, wq2, wv2, wrow)

    qv = Q[0].reshape(_T * _L * _H, _E)
    qr = _sc_gather(qv, flatidx.reshape(_NG)).reshape(_B, _T, _H, _U, _E)

    out0 = pl.pallas_call(
        _fix_kernel,
        grid=(_T, _H // 2),
        in_specs=[
            pl.BlockSpec((1, 1, _L, 128), lambda t, g: (0, t, 0, g)),
            pl.BlockSpec((1, 1, _L, 128), lambda t, g: (1, t, 0, g)),
            pl.BlockSpec((1, 1, _H, _U, _E), lambda t, g: (0, t, 0, 0, 0)),
            pl.BlockSpec((1, 1, _H, _U, _E), lambda t, g: (1, t, 0, 0, 0)),
            pl.BlockSpec((1, 1, 16, 128), lambda t, g: (0, t, 0, 0)),
            pl.BlockSpec((1, 1, 16, 128), lambda t, g: (1, t, 0, 0)),
            pl.BlockSpec((1, 1, _L, _E), lambda t, g: (0, t, 0, 0)),
            pl.BlockSpec(memory_space=pltpu.SMEM),
            pl.BlockSpec(memory_space=pltpu.SMEM),
        ],
        out_specs=pl.BlockSpec((1, _L, _E), lambda t, g: (t, 0, 0)),
        out_shape=jax.ShapeDtypeStruct((_T, _L, _E), jnp.float32),
    )(V, V, qr, qr, topidx, topidx, out_base, w_pad, scl)

    return jnp.concatenate([out0[None], out_base[1:]], axis=0)


# speedup vs baseline: 1.5080x; 1.0476x over previous
import functools

import numpy as np
import jax
import jax.numpy as jnp
from jax.experimental import pallas as pl
from jax.experimental.pallas import tpu as pltpu
from jax.experimental.pallas import tpu_sc as plsc

_B, _T, _L, _D, _H, _E = 2, 4, 2048, 768, 12, 64
_U = 40
_IDX_SAMPLE = np.random.default_rng(0).choice(_L, _U, replace=False)
_CH = 1024
_NC = _L // _CH
_CS = 512
_NEG = float("-inf")
_HIGH = jax.lax.Precision.HIGHEST


def _proj_kernel(q_ref, v_ref, vs_ref, wq_ref, wv_ref, wrow_ref,
                 vout_ref, qout_ref, idx_ref, fi_ref, ob_ref, m_scr,
                 carry_scr, ksbd_scr):
    c = pl.program_id(1)
    prec_m = None

    Qc = jnp.dot(q_ref[0, 0], wq_ref[...], preferred_element_type=jnp.float32,
                 precision=prec_m)
    Vc = jnp.dot(v_ref[0, 0], wv_ref[...], preferred_element_type=jnp.float32,
                 precision=prec_m)
    vout_ref[0, 0] = Vc
    qout_ref[0, 0] = Qc

    @pl.when(c == 0)
    def _():
        carry_scr[...] = jnp.zeros_like(carry_scr)
        Ks = jnp.dot(vs_ref[0, 0], wv_ref[...],
                     preferred_element_type=jnp.float32,
                     precision=prec_m)
        rowsb = jax.lax.broadcasted_iota(jnp.int32, (_H * _U, _H * _E), 0)
        colsb = jax.lax.broadcasted_iota(jnp.int32, (_H * _U, _H * _E), 1)
        Ktile = jnp.concatenate([Ks] * _H, axis=0)
        ksbd_scr[...] = jnp.where(rowsb // _U == colsb // _E, Ktile, 0.0)

    St = jax.lax.dot_general(ksbd_scr[...], Qc, (((1,), (1,)), ((), ())),
                             preferred_element_type=jnp.float32,
                             precision=prec_m)
    for h in range(_H):
        Sh = St[h * _U:(h + 1) * _U, :]
        Mrow = jnp.max(Sh, axis=0) - jnp.sum(Sh, axis=0) * (1.0 / _L)
        m_scr[h, pl.ds(c * _CH, _CH)] = Mrow

    Vw = Vc * wrow_ref[0:1, :]
    R = (Vw[:, 0:128] + Vw[:, 128:256] + Vw[:, 256:384] + Vw[:, 384:512]
         + Vw[:, 512:640] + Vw[:, 640:768])
    Z = R[:, 0:_E] + R[:, _E:128]
    tri = (jax.lax.broadcasted_iota(jnp.int32, (_CS, _CS), 0)
           >= jax.lax.broadcasted_iota(jnp.int32, (_CS, _CS), 1)
           ).astype(jnp.float32)
    carry = carry_scr[0:1, :]
    parts = []
    for s in range(_CH // _CS):
        Zs = Z[s * _CS:(s + 1) * _CS, :]
        parts.append(jnp.dot(tri, Zs, preferred_element_type=jnp.float32,
                             precision=None) + carry)
        carry = carry + jnp.sum(Zs, axis=0, keepdims=True)
    carry_scr[0:1, :] = carry
    ob_ref[0, 0] = jnp.concatenate(parts, axis=0)

    @pl.when(c == _NC - 1)
    def _():
        rows = jax.lax.broadcasted_iota(jnp.int32, (16, _L), 0)
        lanes = jax.lax.broadcasted_iota(jnp.int32, (16, _L), 1)
        cols = jax.lax.broadcasted_iota(jnp.int32, (16, 128), 1)
        M = jnp.where(rows < _H, m_scr[...], _NEG)

        def body(j, state):
            M, acc = state
            mval = jnp.max(M, axis=1, keepdims=True)
            cand = jnp.where(M == mval, lanes, _L)
            idx = jnp.min(cand, axis=1, keepdims=True)
            acc = jnp.where(cols == j, idx, acc)
            M = jnp.where(lanes == idx, _NEG, M)
            return M, acc

        _, acc = jax.lax.fori_loop(0, _U, body,
                                   (M, jnp.zeros((16, 128), jnp.int32)))
        idx_ref[0, 0] = acc
        t = pl.program_id(0) % _T
        rows128 = jax.lax.broadcasted_iota(jnp.int32, (16, 128), 0)
        flat = (t * _L + acc) * _H + rows128
        fi_ref[0, 0] = flat[:_H, :_U]


_NW = 32
_NG = _B * _T * _H * _U
_PER_W = _NG // _NW


def _sc_gather(qv, idxflat):
    mesh = plsc.VectorSubcoreMesh(core_axis_name="c", subcore_axis_name="s")

    @functools.partial(
        pl.kernel, mesh=mesh,
        compiler_params=pltpu.CompilerParams(use_tc_tiling_on_sc=False),
        out_type=jax.ShapeDtypeStruct((_NG, _E), jnp.float32),
        scratch_types=[
            pltpu.VMEM((_PER_W,), jnp.int32),
            pltpu.VMEM((_PER_W, _E), jnp.float32),
            pltpu.SemaphoreType.DMA,
        ],
    )
    def k(q_hbm, idx_hbm, out_hbm, idx_v, rows_v, sem):
        wid = jax.lax.axis_index("s") * 2 + jax.lax.axis_index("c")
        base = wid * _PER_W
        pltpu.sync_copy(idx_hbm.at[pl.ds(base, _PER_W)], idx_v)
        pltpu.async_copy(q_hbm.at[idx_v], rows_v, sem).wait()
        pltpu.sync_copy(rows_v, out_hbm.at[pl.ds(base, _PER_W)])

    return k(qv, idxflat)


def _fix_kernel(v0_ref, v1_ref, qr0_ref, qr1_ref, i0_ref, i1_ref, ob_ref,
                w_ref, s_ref, out_ref):
    V0 = v0_ref[0, 0]
    V1 = v1_ref[0, 0]
    i0 = i0_ref[0, 0]
    i1 = i1_ref[0, 0]
    q0 = qr0_ref[0, 0]
    q1 = qr1_ref[0, 0]
    tau = s_ref[0]
    delta = s_ref[1]
    scale = jnp.float32(1.0 / np.sqrt(_E))
    U2 = 2 * _U
    lanesL2 = jax.lax.broadcasted_iota(jnp.int32, (U2, _L), 1)
    rows2 = jax.lax.broadcasted_iota(jnp.int32, (U2, 128), 0)
    cols2 = jax.lax.broadcasted_iota(jnp.int32, (U2, 128), 1)
    bdmask = ((rows2 < _U) == (cols2 < _E)).astype(jnp.float32)
    same_head = ((jax.lax.broadcasted_iota(jnp.int32, (U2, U2), 0) < _U)
                 == (jax.lax.broadcasted_iota(jnp.int32, (U2, U2), 1) < _U)
                 ).astype(jnp.float32)
    z = jnp.zeros((_U, _E), jnp.float32)

    acc = jnp.zeros((_L, _E), jnp.float32)
    for g in range(_H // 2):
        sl = slice(128 * g, 128 * (g + 1))
        V0p = V0[:, sl]
        V1p = V1[:, sl]
        w2 = jnp.where(rows2[:, 0:1] < _U, w_ref[2 * g], w_ref[2 * g + 1])

        idx2, P2, SM2, Qr2 = [], [], [], []
        for ival, qval in ((i0, q0), (i1, q1)):
            ia = ival[2 * g, :_U].reshape(_U, 1)
            ib = ival[2 * g + 1, :_U].reshape(_U, 1)
            ix = jnp.concatenate([ia, ib], axis=0)
            idx2.append(ix)
            P2.append((lanesL2 == ix).astype(jnp.float32))
            SM2.append((lanesL2 <= ix).astype(jnp.float32))
            Qr2.append(jnp.concatenate([
                jnp.concatenate([qval[2 * g], z], axis=1),
                jnp.concatenate([z, qval[2 * g + 1]], axis=1)],
                axis=0))

        deltas = []
        for b, Vp in enumerate((V0p, V1p)):
            S = jax.lax.dot_general(Qr2[b], Vp, (((1,), (1,)), ((), ())),
                                    preferred_element_type=jnp.float32)
            S = (S * tau + delta) * scale
            S = S - jnp.max(S, axis=1, keepdims=True)
            Sexp = jnp.exp(S)
            A = Sexp / jnp.sum(Sexp, axis=1, keepdims=True)
            attn = jnp.dot(A, Vp, preferred_element_type=jnp.float32)
            cumsel = jnp.dot(SM2[b], V0p, preferred_element_type=jnp.float32)
            deltas.append((attn - cumsel) * bdmask * w2)

        eq = (idx2[0] == idx2[1].reshape(1, U2)).astype(jnp.float32)
        hit = jnp.max(eq * same_head, axis=1, keepdims=True)
        d0 = deltas[0] * (1.0 - hit)

        Pall = jnp.concatenate([P2[0], P2[1]], axis=0)
        dall = jnp.concatenate([d0, deltas[1]], axis=0)
        tpair = jax.lax.dot_general(Pall, dall, (((0,), (0,)), ((), ())),
                                    preferred_element_type=jnp.float32)
        acc = acc + tpair[:, 0:_E] + tpair[:, _E:128]

    out_ref[0] = ob_ref[0, 0] + acc


def kernel(queries, keys, values, Wq, Wk, Wv, w_out, tau, delta):
    del keys, Wk
    wq2 = Wq.reshape(_D, _H * _E)
    wv2 = Wv.reshape(_D, _H * _E)
    vs = values[:, :, _IDX_SAMPLE, :]
    wrow = jnp.broadcast_to(jnp.repeat(w_out, _E)[None, :], (8, _H * _E))
    w_pad = jnp.concatenate([w_out, jnp.zeros((4,), jnp.float32)])
    scl = jnp.concatenate([tau, delta]).astype(jnp.float32)

    def bmap(p):
        return 1 - p // _T

    def tmap(p):
        return p % _T

    V, Q, topidx, flatidx, out_base = pl.pallas_call(
        _proj_kernel,
        grid=(_B * _T, _NC),
        in_specs=[
            pl.BlockSpec((1, 1, _CH, _D), lambda p, c: (bmap(p), tmap(p), c, 0)),
            pl.BlockSpec((1, 1, _CH, _D), lambda p, c: (bmap(p), tmap(p), c, 0)),
            pl.BlockSpec((1, 1, _U, _D), lambda p, c: (bmap(p), tmap(p), 0, 0)),
            pl.BlockSpec((_D, _H * _E), lambda p, c: (0, 0)),
            pl.BlockSpec((_D, _H * _E), lambda p, c: (0, 0)),
            pl.BlockSpec((8, _H * _E), lambda p, c: (0, 0)),
        ],
        out_specs=[
            pl.BlockSpec((1, 1, _CH, _H * _E),
                         lambda p, c: (bmap(p), tmap(p), c, 0)),
            pl.BlockSpec((1, 1, _CH, _H * _E),
                         lambda p, c: (bmap(p), tmap(p), c, 0)),
            pl.BlockSpec((1, 1, 16, 128), lambda p, c: (bmap(p), tmap(p), 0, 0)),
            pl.BlockSpec((1, 1, _H, _U), lambda p, c: (bmap(p), tmap(p), 0, 0)),
            pl.BlockSpec((1, 1, _CH, _E), lambda p, c: (bmap(p), tmap(p), c, 0)),
        ],
        out_shape=[
            jax.ShapeDtypeStruct((_B, _T, _L, _H * _E), jnp.float32),
            jax.ShapeDtypeStruct((_B, _T, _L, _H * _E), jnp.float32),
            jax.ShapeDtypeStruct((_B, _T, 16, 128), jnp.int32),
            jax.ShapeDtypeStruct((_B, _T, _H, _U), jnp.int32),
            jax.ShapeDtypeStruct((_B, _T, _L, _E), jnp.float32),
        ],
        scratch_shapes=[
            pltpu.VMEM((16, _L), jnp.float32),
            pltpu.VMEM((8, _E), jnp.float32),
            pltpu.VMEM((_H * _U, _H * _E), jnp.float32),
        ],
    )(queries, values, vs, wq2, wv2, wrow)

    qv = Q[0].reshape(_T * _L * _H, _E)
    qr = _sc_gather(qv, flatidx.reshape(_NG)).reshape(_B, _T, _H, _U, _E)

    out0 = pl.pallas_call(
        _fix_kernel,
        grid=(_T,),
        in_specs=[
            pl.BlockSpec((1, 1, _L, _H * _E), lambda t: (0, t, 0, 0)),
            pl.BlockSpec((1, 1, _L, _H * _E), lambda t: (1, t, 0, 0)),
            pl.BlockSpec((1, 1, _H, _U, _E), lambda t: (0, t, 0, 0, 0)),
            pl.BlockSpec((1, 1, _H, _U, _E), lambda t: (1, t, 0, 0, 0)),
            pl.BlockSpec((1, 1, 16, 128), lambda t: (0, t, 0, 0)),
            pl.BlockSpec((1, 1, 16, 128), lambda t: (1, t, 0, 0)),
            pl.BlockSpec((1, 1, _L, _E), lambda t: (0, t, 0, 0)),
            pl.BlockSpec(memory_space=pltpu.SMEM),
            pl.BlockSpec(memory_space=pltpu.SMEM),
        ],
        out_specs=pl.BlockSpec((1, _L, _E), lambda t: (t, 0, 0)),
        out_shape=jax.ShapeDtypeStruct((_T, _L, _E), jnp.float32),
    )(V, V, qr, qr, topidx, topidx, out_base, w_pad, scl)

    return jnp.concatenate([out0[None], out_base[1:]], axis=0)


# speedup vs baseline: 1.5358x; 1.0184x over previous
import functools

import numpy as np
import jax
import jax.numpy as jnp
from jax.experimental import pallas as pl
from jax.experimental.pallas import tpu as pltpu
from jax.experimental.pallas import tpu_sc as plsc

_B, _T, _L, _D, _H, _E = 2, 4, 2048, 768, 12, 64
_U = 40
_IDX_SAMPLE = np.random.default_rng(0).choice(_L, _U, replace=False)
_CH = 1024
_NC = _L // _CH
_CS = 512
_NEG = float("-inf")


def _proj_kernel(q_ref, v_ref, vs_ref, wq_ref, wv_ref, wrow_ref,
                 vout_ref, qout_ref, idx_ref, fi_ref, ob_ref, m_scr,
                 carry_scr, ksbd_scr, qtmp_scr, qsem):
    p = pl.program_id(0)
    c = pl.program_id(1)
    prec_m = None

    Qc = jnp.dot(q_ref[0, 0], wq_ref[...], preferred_element_type=jnp.float32,
                 precision=prec_m)
    Vc = jnp.dot(v_ref[0, 0], wv_ref[...], preferred_element_type=jnp.float32,
                 precision=prec_m)
    vout_ref[0, 0] = Vc

    @pl.when(p >= _T)
    def _():
        qtmp_scr[...] = Qc
        t = p % _T
        cp = pltpu.make_async_copy(
            qtmp_scr, qout_ref.at[t, pl.ds(c * _CH, _CH), :], qsem)
        cp.start()
        cp.wait()

    @pl.when(c == 0)
    def _():
        carry_scr[...] = jnp.zeros_like(carry_scr)
        Ks = jnp.dot(vs_ref[0, 0], wv_ref[...],
                     preferred_element_type=jnp.float32,
                     precision=prec_m)
        rowsb = jax.lax.broadcasted_iota(jnp.int32, (_H * _U, _H * _E), 0)
        colsb = jax.lax.broadcasted_iota(jnp.int32, (_H * _U, _H * _E), 1)
        Ktile = jnp.concatenate([Ks] * _H, axis=0)
        ksbd_scr[...] = jnp.where(rowsb // _U == colsb // _E, Ktile, 0.0)

    St = jax.lax.dot_general(ksbd_scr[...], Qc, (((1,), (1,)), ((), ())),
                             preferred_element_type=jnp.float32,
                             precision=prec_m)
    for h in range(_H):
        Sh = St[h * _U:(h + 1) * _U, :]
        Mrow = jnp.max(Sh, axis=0) - jnp.sum(Sh, axis=0) * (1.0 / _L)
        m_scr[h, pl.ds(c * _CH, _CH)] = Mrow

    Vw = Vc * wrow_ref[0:1, :]
    R = (Vw[:, 0:128] + Vw[:, 128:256] + Vw[:, 256:384] + Vw[:, 384:512]
         + Vw[:, 512:640] + Vw[:, 640:768])
    Z = R[:, 0:_E] + R[:, _E:128]
    tri = (jax.lax.broadcasted_iota(jnp.int32, (_CS, _CS), 0)
           >= jax.lax.broadcasted_iota(jnp.int32, (_CS, _CS), 1)
           ).astype(jnp.float32)
    carry = carry_scr[0:1, :]
    parts = []
    for s in range(_CH // _CS):
        Zs = Z[s * _CS:(s + 1) * _CS, :]
        parts.append(jnp.dot(tri, Zs, preferred_element_type=jnp.float32,
                             precision=None) + carry)
        carry = carry + jnp.sum(Zs, axis=0, keepdims=True)
    carry_scr[0:1, :] = carry
    ob_ref[0, 0] = jnp.concatenate(parts, axis=0)

    @pl.when(c == _NC - 1)
    def _():
        rows = jax.lax.broadcasted_iota(jnp.int32, (16, _L), 0)
        lanes = jax.lax.broadcasted_iota(jnp.int32, (16, _L), 1)
        cols = jax.lax.broadcasted_iota(jnp.int32, (16, 128), 1)
        M = jnp.where(rows < _H, m_scr[...], _NEG)

        def body(j, state):
            M, acc = state
            mval = jnp.max(M, axis=1, keepdims=True)
            cand = jnp.where(M == mval, lanes, _L)
            idx = jnp.min(cand, axis=1, keepdims=True)
            acc = jnp.where(cols == j, idx, acc)
            M = jnp.where(lanes == idx, _NEG, M)
            return M, acc

        _, acc = jax.lax.fori_loop(0, _U, body,
                                   (M, jnp.zeros((16, 128), jnp.int32)))
        idx_ref[0, 0] = acc
        t = pl.program_id(0) % _T
        rows128 = jax.lax.broadcasted_iota(jnp.int32, (16, 128), 0)
        flat = (t * _L + acc) * _H + rows128
        fi_ref[0, 0] = flat[:_H, :_U]


_NW = 32
_NG = _B * _T * _H * _U
_PER_W = _NG // _NW


def _sc_gather(qv, idxflat):
    mesh = plsc.VectorSubcoreMesh(core_axis_name="c", subcore_axis_name="s")

    @functools.partial(
        pl.kernel, mesh=mesh,
        compiler_params=pltpu.CompilerParams(use_tc_tiling_on_sc=False),
        out_type=jax.ShapeDtypeStruct((_NG, _E), jnp.float32),
        scratch_types=[
            pltpu.VMEM((_PER_W,), jnp.int32),
            pltpu.VMEM((_PER_W, _E), jnp.float32),
            pltpu.SemaphoreType.DMA,
        ],
    )
    def k(q_hbm, idx_hbm, out_hbm, idx_v, rows_v, sem):
        wid = jax.lax.axis_index("s") * 2 + jax.lax.axis_index("c")
        base = wid * _PER_W
        pltpu.sync_copy(idx_hbm.at[pl.ds(base, _PER_W)], idx_v)
        pltpu.async_copy(q_hbm.at[idx_v], rows_v, sem).wait()
        pltpu.sync_copy(rows_v, out_hbm.at[pl.ds(base, _PER_W)])

    return k(qv, idxflat)


def _fix_kernel(v0_ref, v1_ref, qr0_ref, qr1_ref, i0_ref, i1_ref, ob_ref,
                w_ref, s_ref, out_ref):
    V0 = v0_ref[0, 0]
    V1 = v1_ref[0, 0]
    i0 = i0_ref[0, 0]
    i1 = i1_ref[0, 0]
    q0 = qr0_ref[0, 0]
    q1 = qr1_ref[0, 0]
    tau = s_ref[0]
    delta = s_ref[1]
    scale = jnp.float32(1.0 / np.sqrt(_E))
    U2 = 2 * _U
    lanesL2 = jax.lax.broadcasted_iota(jnp.int32, (U2, _L), 1)
    rows2 = jax.lax.broadcasted_iota(jnp.int32, (U2, 128), 0)
    cols2 = jax.lax.broadcasted_iota(jnp.int32, (U2, 128), 1)
    bdmask = ((rows2 < _U) == (cols2 < _E)).astype(jnp.float32)
    same_head = ((jax.lax.broadcasted_iota(jnp.int32, (U2, U2), 0) < _U)
                 == (jax.lax.broadcasted_iota(jnp.int32, (U2, U2), 1) < _U)
                 ).astype(jnp.float32)
    z = jnp.zeros((_U, _E), jnp.float32)

    acc = jnp.zeros((_L, _E), jnp.float32)
    for g in range(_H // 2):
        sl = slice(128 * g, 128 * (g + 1))
        V0p = V0[:, sl]
        V1p = V1[:, sl]
        w2 = jnp.where(rows2[:, 0:1] < _U, w_ref[2 * g], w_ref[2 * g + 1])

        idx2, P2, SM2, Qr2 = [], [], [], []
        for ival, qval in ((i0, q0), (i1, q1)):
            ia = ival[2 * g, :_U].reshape(_U, 1)
            ib = ival[2 * g + 1, :_U].reshape(_U, 1)
            ix = jnp.concatenate([ia, ib], axis=0)
            idx2.append(ix)
            P2.append((lanesL2 == ix).astype(jnp.float32))
            SM2.append((lanesL2 <= ix).astype(jnp.float32))
            Qr2.append(jnp.concatenate([
                jnp.concatenate([qval[2 * g], z], axis=1),
                jnp.concatenate([z, qval[2 * g + 1]], axis=1)],
                axis=0))

        deltas = []
        for b, Vp in enumerate((V0p, V1p)):
            S = jax.lax.dot_general(Qr2[b], Vp, (((1,), (1,)), ((), ())),
                                    preferred_element_type=jnp.float32)
            S = (S * tau + delta) * scale
            S = S - jnp.max(S, axis=1, keepdims=True)
            Sexp = jnp.exp(S)
            A = Sexp / jnp.sum(Sexp, axis=1, keepdims=True)
            attn = jnp.dot(A, Vp, preferred_element_type=jnp.float32)
            cumsel = jnp.dot(SM2[b], V0p, preferred_element_type=jnp.float32)
            deltas.append((attn - cumsel) * bdmask * w2)

        eq = (idx2[0] == idx2[1].reshape(1, U2)).astype(jnp.float32)
        hit = jnp.max(eq * same_head, axis=1, keepdims=True)
        d0 = deltas[0] * (1.0 - hit)

        Pall = jnp.concatenate([P2[0], P2[1]], axis=0)
        dall = jnp.concatenate([d0, deltas[1]], axis=0)
        tpair = jax.lax.dot_general(Pall, dall, (((0,), (0,)), ((), ())),
                                    preferred_element_type=jnp.float32)
        acc = acc + tpair[:, 0:_E] + tpair[:, _E:128]

    out_ref[0] = ob_ref[0, 0] + acc


def kernel(queries, keys, values, Wq, Wk, Wv, w_out, tau, delta):
    del keys, Wk
    wq2 = Wq.reshape(_D, _H * _E)
    wv2 = Wv.reshape(_D, _H * _E)
    vs = values[:, :, _IDX_SAMPLE, :]
    wrow = jnp.broadcast_to(jnp.repeat(w_out, _E)[None, :], (8, _H * _E))
    w_pad = jnp.concatenate([w_out, jnp.zeros((4,), jnp.float32)])
    scl = jnp.concatenate([tau, delta]).astype(jnp.float32)

    def bmap(p):
        return 1 - p // _T

    def tmap(p):
        return p % _T

    V, Q, topidx, flatidx, out_base = pl.pallas_call(
        _proj_kernel,
        grid=(_B * _T, _NC),
        in_specs=[
            pl.BlockSpec((1, 1, _CH, _D), lambda p, c: (bmap(p), tmap(p), c, 0)),
            pl.BlockSpec((1, 1, _CH, _D), lambda p, c: (bmap(p), tmap(p), c, 0)),
            pl.BlockSpec((1, 1, _U, _D), lambda p, c: (bmap(p), tmap(p), 0, 0)),
            pl.BlockSpec((_D, _H * _E), lambda p, c: (0, 0)),
            pl.BlockSpec((_D, _H * _E), lambda p, c: (0, 0)),
            pl.BlockSpec((8, _H * _E), lambda p, c: (0, 0)),
        ],
        out_specs=[
            pl.BlockSpec((1, 1, _CH, _H * _E),
                         lambda p, c: (bmap(p), tmap(p), c, 0)),
            pl.BlockSpec(memory_space=pl.ANY),
            pl.BlockSpec((1, 1, 16, 128), lambda p, c: (bmap(p), tmap(p), 0, 0)),
            pl.BlockSpec((1, 1, _H, _U), lambda p, c: (bmap(p), tmap(p), 0, 0)),
            pl.BlockSpec((1, 1, _CH, _E), lambda p, c: (bmap(p), tmap(p), c, 0)),
        ],
        out_shape=[
            jax.ShapeDtypeStruct((_B, _T, _L, _H * _E), jnp.float32),
            jax.ShapeDtypeStruct((_T, _L, _H * _E), jnp.float32),
            jax.ShapeDtypeStruct((_B, _T, 16, 128), jnp.int32),
            jax.ShapeDtypeStruct((_B, _T, _H, _U), jnp.int32),
            jax.ShapeDtypeStruct((_B, _T, _L, _E), jnp.float32),
        ],
        scratch_shapes=[
            pltpu.VMEM((16, _L), jnp.float32),
            pltpu.VMEM((8, _E), jnp.float32),
            pltpu.VMEM((_H * _U, _H * _E), jnp.float32),
            pltpu.VMEM((_CH, _H * _E), jnp.float32),
            pltpu.SemaphoreType.DMA,
        ],
    )(queries, values, vs, wq2, wv2, wrow)

    qv = Q.reshape(_T * _L * _H, _E)
    qr = _sc_gather(qv, flatidx.reshape(_NG)).reshape(_B, _T, _H, _U, _E)

    out0 = pl.pallas_call(
        _fix_kernel,
        grid=(_T,),
        in_specs=[
            pl.BlockSpec((1, 1, _L, _H * _E), lambda t: (0, t, 0, 0)),
            pl.BlockSpec((1, 1, _L, _H * _E), lambda t: (1, t, 0, 0)),
            pl.BlockSpec((1, 1, _H, _U, _E), lambda t: (0, t, 0, 0, 0)),
            pl.BlockSpec((1, 1, _H, _U, _E), lambda t: (1, t, 0, 0, 0)),
            pl.BlockSpec((1, 1, 16, 128), lambda t: (0, t, 0, 0)),
            pl.BlockSpec((1, 1, 16, 128), lambda t: (1, t, 0, 0)),
            pl.BlockSpec((1, 1, _L, _E), lambda t: (0, t, 0, 0)),
            pl.BlockSpec(memory_space=pltpu.SMEM),
            pl.BlockSpec(memory_space=pltpu.SMEM),
        ],
        out_specs=pl.BlockSpec((1, _L, _E), lambda t: (t, 0, 0)),
        out_shape=jax.ShapeDtypeStruct((_T, _L, _E), jnp.float32),
    )(V, V, qr, qr, topidx, topidx, out_base, w_pad, scl)

    return jnp.concatenate([out0[None], out_base[1:]], axis=0)
